# baseline (speedup 1.0000x reference)
import functools

import jax
import jax.numpy as jnp
from jax.experimental import pallas as pl
from jax.experimental.pallas import tpu as pltpu

_K = 8192
_H = _K // 2
_D = 32
_N = 16 * 32 * 32
_R = 256
_G = _N // _R
_COMMITMENT_COST = 0.25


def _vq_kernel(x2b_ref, z_ref, xn_ref, cn_ref, cb_ref,
               q_ref, idx_ref, loss_ref, perp_ref, counts_ref):
    i = pl.program_id(0)
    x2 = x2b_ref[...]
    zv = z_ref[...]
    cb = cb_ref[...]

    mm = jax.lax.dot_general(x2, cb, (((1,), (1,)), ((), ())),
                             preferred_element_type=jnp.float32)
    dist = (xn_ref[...] + cn_ref[...]) - mm

    dl = dist[:, :_H]
    dh = dist[:, _H:]
    ml = jnp.min(dl, axis=1, keepdims=True)
    mh = jnp.min(dh, axis=1, keepdims=True)
    hiota = jax.lax.broadcasted_iota(jnp.int32, (_R, _H), 1)
    il = jnp.min(jnp.where(dl <= ml, hiota, _K), axis=1)
    ih = jnp.min(jnp.where(dh <= mh, hiota, _K), axis=1) + _H

    vl_r = ml[:, 0].astype(jnp.bfloat16).astype(jnp.float32)
    vh_bits = jax.lax.bitcast_convert_type(mh[:, 0], jnp.int32)
    vh_t = jax.lax.bitcast_convert_type(
        jnp.bitwise_and(vh_bits, jnp.int32(-65536)), jnp.float32)
    idx = jnp.where(vh_t < vl_r, ih, il)
    idx_ref[...] = idx

    kiota = jax.lax.broadcasted_iota(jnp.int32, (_R, _K), 1)
    onehot = (kiota == idx[:, None]).astype(jnp.float32)
    q = jax.lax.dot_general(onehot, cb, (((1,), (0,)), ((), ())),
                            precision=jax.lax.Precision.HIGHEST,
                            preferred_element_type=jnp.float32)
    q_ref[...] = zv + (q - zv)

    @pl.when(i == 0)
    def _init():
        counts_ref[...] = jnp.zeros_like(counts_ref)
        loss_ref[0, 0] = 0.0
        perp_ref[0, 0] = 0.0

    counts_ref[...] += jnp.sum(onehot, axis=0)
    loss_ref[0, 0] += jnp.sum((q - zv) ** 2)

    @pl.when(i == _G - 1)
    def _fini():
        loss_ref[0, 0] = loss_ref[0, 0] * (_COMMITMENT_COST / (_N * _D))
        p = counts_ref[...] * (1.0 / _N)
        perp_ref[0, 0] = jnp.exp(-jnp.sum(p * jnp.log(p + 1e-10)))


@functools.partial(jax.jit, static_argnames=("interpret",))
def kernel(inputs, codebook, interpret=False):
    z = jnp.transpose(inputs, (0, 2, 3, 1))
    flat = z.reshape(_N, _D)
    xnorm = jnp.sum(flat ** 2, axis=1, keepdims=True)
    cnorm = jnp.sum(codebook ** 2, axis=1)[None, :]
    x2b = (2.0 * flat).astype(jnp.bfloat16)

    q_flat, idx, loss, perp = pl.pallas_call(
        _vq_kernel,
        grid=(_G,),
        in_specs=[
            pl.BlockSpec((_R, _D), lambda i: (i, 0)),
            pl.BlockSpec((_R, _D), lambda i: (i, 0)),
            pl.BlockSpec((_R, 1), lambda i: (i, 0)),
            pl.BlockSpec((1, _K), lambda i: (0, 0)),
            pl.BlockSpec((_K, _D), lambda i: (0, 0)),
        ],
        out_specs=[
            pl.BlockSpec((_R, _D), lambda i: (i, 0)),
            pl.BlockSpec((_R,), lambda i: (i,)),
            pl.BlockSpec(memory_space=pltpu.SMEM),
            pl.BlockSpec(memory_space=pltpu.SMEM),
        ],
        out_shape=[
            jax.ShapeDtypeStruct((_N, _D), jnp.float32),
            jax.ShapeDtypeStruct((_N,), jnp.int32),
            jax.ShapeDtypeStruct((1, 1), jnp.float32),
            jax.ShapeDtypeStruct((1, 1), jnp.float32),
        ],
        scratch_shapes=[pltpu.VMEM((_K,), jnp.float32)],
        interpret=interpret,
    )(x2b, flat, xnorm, cnorm, codebook)

    quantized_out = jnp.transpose(q_flat.reshape(16, 32, 32, _D), (0, 3, 1, 2))
    return quantized_out, loss[0, 0], perp[0, 0], idx


# speedup vs baseline: 1.6907x; 1.6907x over previous
import functools

import jax
import jax.numpy as jnp
from jax.experimental import pallas as pl
from jax.experimental.pallas import tpu as pltpu

_K = 8192
_H = _K // 2
_D = 32
_N = 16 * 32 * 32
_R = 256
_G = _N // _R
_COMMITMENT_COST = 0.25


def _vq_kernel(x2b_ref, z_ref, xn_ref, cn_ref, cb_ref, cbh_ref, cbm_ref, cbl_ref,
               q_ref, idx_ref, loss_ref, perp_ref, counts_ref):
    i = pl.program_id(0)
    x2 = x2b_ref[...]
    zv = z_ref[...]
    cb = cb_ref[...]

    mm = jax.lax.dot_general(x2, cb, (((1,), (1,)), ((), ())),
                             preferred_element_type=jnp.float32)
    dist = (xn_ref[...] + cn_ref[...]) - mm

    dl = dist[:, :_H]
    dh = dist[:, _H:]
    ml = jnp.min(dl, axis=1)
    mh = jnp.min(dh, axis=1)
    il = jnp.argmin(dl, axis=1)
    ih = jnp.argmin(dh, axis=1) + _H

    vl_r = ml.astype(jnp.bfloat16).astype(jnp.float32)
    vh_bits = jax.lax.bitcast_convert_type(mh, jnp.int32)
    vh_t = jax.lax.bitcast_convert_type(
        jnp.bitwise_and(vh_bits, jnp.int32(-65536)), jnp.float32)
    take_hi = vh_t < vl_r
    idx = jnp.where(take_hi, ih, il)
    idx_ref[...] = idx

    kiota = jax.lax.broadcasted_iota(jnp.int32, (_R, _K), 1)
    onehot = (kiota == idx[:, None]).astype(jnp.float32)
    dn = (((1,), (0,)), ((), ()))
    ohb = onehot.astype(jnp.bfloat16)
    q = (jax.lax.dot_general(ohb, cbh_ref[...], dn,
                             preferred_element_type=jnp.float32)
         + jax.lax.dot_general(ohb, cbm_ref[...], dn,
                               preferred_element_type=jnp.float32)
         + jax.lax.dot_general(ohb, cbl_ref[...], dn,
                               preferred_element_type=jnp.float32))
    q_ref[...] = zv + (q - zv)

    @pl.when(i == 0)
    def _init():
        counts_ref[...] = jnp.zeros_like(counts_ref)
        loss_ref[0, 0] = 0.0
        perp_ref[0, 0] = 0.0

    counts_ref[...] += jnp.sum(onehot, axis=0)
    loss_ref[0, 0] += jnp.sum((q - zv) ** 2)

    @pl.when(i == _G - 1)
    def _fini():
        loss_ref[0, 0] = loss_ref[0, 0] * (_COMMITMENT_COST / (_N * _D))
        p = counts_ref[...] * (1.0 / _N)
        perp_ref[0, 0] = jnp.exp(-jnp.sum(p * jnp.log(p + 1e-10)))


@functools.partial(jax.jit, static_argnames=("interpret",))
def kernel(inputs, codebook, interpret=False):
    z = jnp.transpose(inputs, (0, 2, 3, 1))
    flat = z.reshape(_N, _D)
    xnorm = jnp.sum(flat ** 2, axis=1, keepdims=True)
    cnorm = jnp.sum(codebook ** 2, axis=1)[None, :]
    x2b = (2.0 * flat).astype(jnp.bfloat16)
    cb_hi = codebook.astype(jnp.bfloat16)
    r1 = codebook - cb_hi.astype(jnp.float32)
    cb_mid = r1.astype(jnp.bfloat16)
    cb_lo = (r1 - cb_mid.astype(jnp.float32)).astype(jnp.bfloat16)

    q_flat, idx, loss, perp = pl.pallas_call(
        _vq_kernel,
        grid=(_G,),
        in_specs=[
            pl.BlockSpec((_R, _D), lambda i: (i, 0)),
            pl.BlockSpec((_R, _D), lambda i: (i, 0)),
            pl.BlockSpec((_R, 1), lambda i: (i, 0)),
            pl.BlockSpec((1, _K), lambda i: (0, 0)),
            pl.BlockSpec((_K, _D), lambda i: (0, 0)),
            pl.BlockSpec((_K, _D), lambda i: (0, 0)),
            pl.BlockSpec((_K, _D), lambda i: (0, 0)),
            pl.BlockSpec((_K, _D), lambda i: (0, 0)),
        ],
        out_specs=[
            pl.BlockSpec((_R, _D), lambda i: (i, 0)),
            pl.BlockSpec((_R,), lambda i: (i,)),
            pl.BlockSpec(memory_space=pltpu.SMEM),
            pl.BlockSpec(memory_space=pltpu.SMEM),
        ],
        out_shape=[
            jax.ShapeDtypeStruct((_N, _D), jnp.float32),
            jax.ShapeDtypeStruct((_N,), jnp.int32),
            jax.ShapeDtypeStruct((1, 1), jnp.float32),
            jax.ShapeDtypeStruct((1, 1), jnp.float32),
        ],
        scratch_shapes=[pltpu.VMEM((_K,), jnp.float32)],
        interpret=interpret,
    )(x2b, flat, xnorm, cnorm, codebook, cb_hi, cb_mid, cb_lo)

    quantized_out = jnp.transpose(q_flat.reshape(16, 32, 32, _D), (0, 3, 1, 2))
    return quantized_out, loss[0, 0], perp[0, 0], idx
